# baseline (speedup 1.0000x reference)
import functools

import jax
import jax.numpy as jnp
from jax import lax
from jax.experimental import pallas as pl
from jax.experimental.pallas import tpu as pltpu
from jax.experimental.pallas import tpu_sc as plsc

_K = 8192
_D = 256
_N = 16384
_BM = 256
_G = _N // _BM
_COMMITMENT_COST = 0.25


_W = 2736


def _dist_body(sz_ref, se_ref, z_ref, emb_ref, codes_ref, minv_ref):
    z = z_ref[...]
    e = emb_ref[...]
    mm = lax.dot_general(
        z.astype(jnp.bfloat16), e.astype(jnp.bfloat16),
        dimension_numbers=(((1,), (1,)), ((), ())),
        preferred_element_type=jnp.float32)
    dist = (sz_ref[...] + se_ref[...]) - 2.0 * mm
    iota = lax.broadcasted_iota(jnp.int32, (_BM, _K), 1)
    acc_v = None
    acc_i = None
    for lo in range(0, _K, _W):
        hi = min(lo + _W, _K)
        dw = lax.slice(dist, (0, lo), (_BM, hi))
        iw = lax.slice(iota, (0, lo), (_BM, hi))
        m = jnp.min(dw, axis=1, keepdims=True)
        idx = jnp.min(jnp.where(dw == m, iw, _K), axis=1, keepdims=True)
        if acc_v is None:
            acc_v, acc_i = m, idx
        else:
            keep = acc_v <= m
            acc_i = jnp.where(keep, acc_i, idx)
            acc_v = jnp.where(keep, acc_v, m)
        acc_v = acc_v.astype(jnp.bfloat16).astype(jnp.float32)
    codes_ref[0, 0, :] = acc_i[:, 0]
    minv = jnp.min(jnp.where(iota == acc_i, dist, jnp.inf), axis=1)
    minv_ref[0, 0, :] = minv


def _argmin_distances(sz, se2d, z_flat, embedding):
    return pl.pallas_call(
        _dist_body,
        grid=(_G,),
        in_specs=[
            pl.BlockSpec((_BM, 1), lambda i: (i, 0)),
            pl.BlockSpec((1, _K), lambda i: (0, 0)),
            pl.BlockSpec((_BM, _D), lambda i: (i, 0)),
            pl.BlockSpec((_K, _D), lambda i: (0, 0)),
        ],
        out_specs=[
            pl.BlockSpec((1, 1, _BM), lambda i: (i, 0, 0)),
            pl.BlockSpec((1, 1, _BM), lambda i: (i, 0, 0)),
        ],
        out_shape=[
            jax.ShapeDtypeStruct((_G, 1, _BM), jnp.int32),
            jax.ShapeDtypeStruct((_G, 1, _BM), jnp.float32),
        ],
    )(sz, se2d, z_flat, embedding)


@functools.lru_cache(maxsize=1)
def _make_sc_gather():
    info = plsc.get_sparse_core_info()
    nw = info.num_cores * info.num_subcores
    b_per_w = _N // nw
    ch = 128
    n_ch = b_per_w // ch
    mesh = plsc.VectorSubcoreMesh(core_axis_name="c", subcore_axis_name="s")

    @functools.partial(
        pl.kernel, mesh=mesh,
        out_type=jax.ShapeDtypeStruct((_N, _D), jnp.float32),
        scratch_types=[
            pltpu.VMEM((ch,), jnp.int32),
            pltpu.VMEM((ch, _D), jnp.float32),
            pltpu.SemaphoreType.DMA,
        ],
    )
    def gather_rows(emb_hbm, codes_hbm, out_hbm, idx_v, rows_v, sem):
        wid = lax.axis_index("s") * info.num_cores + lax.axis_index("c")
        base = wid * b_per_w
        for c in range(n_ch):
            off = base + c * ch
            pltpu.sync_copy(codes_hbm.at[pl.ds(off, ch)], idx_v)
            pltpu.async_copy(emb_hbm.at[idx_v], rows_v, sem).wait()
            pltpu.sync_copy(rows_v, out_hbm.at[pl.ds(off, ch)])

    return gather_rows


def kernel(z, embedding):
    z_flat = z.reshape(-1, _D)
    sz = jnp.sum(z_flat ** 2, axis=1, keepdims=True)
    se = jnp.sum(embedding ** 2, axis=1)
    codes3, minv3 = _argmin_distances(sz, se.reshape(1, _K), z_flat, embedding)
    codes = codes3.reshape(_N)
    z_q = _make_sc_gather()(embedding, codes).reshape(z.shape)
    vq_loss = _COMMITMENT_COST * (jnp.sum(minv3) / (_N * _D))
    return (vq_loss, z_q, codes.reshape(z.shape[0], -1))


# speedup vs baseline: 1.2006x; 1.2006x over previous
import functools

import jax
import jax.numpy as jnp
from jax import lax
from jax.experimental import pallas as pl
from jax.experimental.pallas import tpu as pltpu
from jax.experimental.pallas import tpu_sc as plsc

_K = 8192
_D = 256
_N = 16384
_BM = 256
_G = _N // _BM
_COMMITMENT_COST = 0.25


_W = 2736


def _dist_body(sz_ref, se_ref, z_ref, emb_ref, codes_ref, minv_ref):
    z = z_ref[...]
    e = emb_ref[...]
    mm = lax.dot_general(
        z.astype(jnp.bfloat16), e,
        dimension_numbers=(((1,), (1,)), ((), ())),
        preferred_element_type=jnp.float32)
    dist = (sz_ref[...] + se_ref[...]) - 2.0 * mm
    iota = lax.broadcasted_iota(jnp.int32, (_BM, _K), 1)
    acc_v = None
    acc_x = None
    acc_i = None
    for lo in range(0, _K, _W):
        hi = min(lo + _W, _K)
        dw = lax.slice(dist, (0, lo), (_BM, hi))
        iw = lax.slice(iota, (0, lo), (_BM, hi))
        m = jnp.min(dw, axis=1, keepdims=True)
        idx = jnp.min(jnp.where(dw == m, iw, _K), axis=1, keepdims=True)
        if acc_v is None:
            acc_v, acc_x, acc_i = m, m, idx
        else:
            keep = acc_v <= m
            acc_i = jnp.where(keep, acc_i, idx)
            acc_x = jnp.where(keep, acc_x, m)
            acc_v = jnp.where(keep, acc_v, m)
        acc_v = acc_v.astype(jnp.bfloat16).astype(jnp.float32)
    codes_ref[0, 0, :] = acc_i[:, 0]
    minv_ref[0, 0, :] = acc_x[:, 0]


def _argmin_distances(sz, se2d, z_flat, embedding):
    return pl.pallas_call(
        _dist_body,
        grid=(_G,),
        in_specs=[
            pl.BlockSpec((_BM, 1), lambda i: (i, 0)),
            pl.BlockSpec((1, _K), lambda i: (0, 0)),
            pl.BlockSpec((_BM, _D), lambda i: (i, 0)),
            pl.BlockSpec((_K, _D), lambda i: (0, 0)),
        ],
        out_specs=[
            pl.BlockSpec((1, 1, _BM), lambda i: (i, 0, 0)),
            pl.BlockSpec((1, 1, _BM), lambda i: (i, 0, 0)),
        ],
        out_shape=[
            jax.ShapeDtypeStruct((_G, 1, _BM), jnp.int32),
            jax.ShapeDtypeStruct((_G, 1, _BM), jnp.float32),
        ],
    )(sz, se2d, z_flat, embedding)


@functools.lru_cache(maxsize=1)
def _make_sc_gather():
    info = plsc.get_sparse_core_info()
    nw = info.num_cores * info.num_subcores
    b_per_w = _N // nw
    ch = 128
    n_ch = b_per_w // ch
    mesh = plsc.VectorSubcoreMesh(core_axis_name="c", subcore_axis_name="s")

    @functools.partial(
        pl.kernel, mesh=mesh,
        out_type=jax.ShapeDtypeStruct((_N, _D), jnp.float32),
        scratch_types=[
            pltpu.VMEM((ch,), jnp.int32),
            pltpu.VMEM((ch, _D), jnp.float32),
            pltpu.SemaphoreType.DMA,
        ],
    )
    def gather_rows(emb_hbm, codes_hbm, out_hbm, idx_v, rows_v, sem):
        wid = lax.axis_index("s") * info.num_cores + lax.axis_index("c")
        base = wid * b_per_w
        for c in range(n_ch):
            off = base + c * ch
            pltpu.sync_copy(codes_hbm.at[pl.ds(off, ch)], idx_v)
            pltpu.async_copy(emb_hbm.at[idx_v], rows_v, sem).wait()
            pltpu.sync_copy(rows_v, out_hbm.at[pl.ds(off, ch)])

    return gather_rows


def kernel(z, embedding):
    z_flat = z.reshape(-1, _D)
    sz = jnp.sum(z_flat ** 2, axis=1, keepdims=True)
    se = jnp.sum(embedding ** 2, axis=1)
    codes3, minv3 = _argmin_distances(
        sz, se.reshape(1, _K), z_flat, embedding.astype(jnp.bfloat16))
    codes = codes3.reshape(_N)
    z_q = _make_sc_gather()(embedding, codes).reshape(z.shape)
    vq_loss = _COMMITMENT_COST * (jnp.sum(minv3) / (_N * _D))
    return (vq_loss, z_q, codes.reshape(z.shape[0], -1))


# speedup vs baseline: 1.2644x; 1.0531x over previous
import functools

import jax
import jax.numpy as jnp
from jax import lax
from jax.experimental import pallas as pl
from jax.experimental.pallas import tpu as pltpu
from jax.experimental.pallas import tpu_sc as plsc

_K = 8192
_D = 256
_N = 16384
_BM = 512
_G = _N // _BM
_COMMITMENT_COST = 0.25


_W = 2736


def _dist_body(sz_ref, se_ref, z_ref, emb_ref, codes_ref, minv_ref):
    z = z_ref[...]
    e = emb_ref[...]
    mm = lax.dot_general(
        z.astype(jnp.bfloat16), e,
        dimension_numbers=(((1,), (1,)), ((), ())),
        preferred_element_type=jnp.float32)
    dist = (sz_ref[...] + se_ref[...]) - 2.0 * mm
    iota = lax.broadcasted_iota(jnp.int32, (_BM, _K), 1)
    acc_v = None
    acc_x = None
    acc_i = None
    for lo in range(0, _K, _W):
        hi = min(lo + _W, _K)
        dw = lax.slice(dist, (0, lo), (_BM, hi))
        iw = lax.slice(iota, (0, lo), (_BM, hi))
        m = jnp.min(dw, axis=1, keepdims=True)
        idx = jnp.min(jnp.where(dw == m, iw, _K), axis=1, keepdims=True)
        if acc_v is None:
            acc_v, acc_x, acc_i = m, m, idx
        else:
            keep = acc_v <= m
            acc_i = jnp.where(keep, acc_i, idx)
            acc_x = jnp.where(keep, acc_x, m)
            acc_v = jnp.where(keep, acc_v, m)
        acc_v = acc_v.astype(jnp.bfloat16).astype(jnp.float32)
    codes_ref[0, 0, :] = acc_i[:, 0]
    minv_ref[0, 0, :] = acc_x[:, 0]


def _argmin_distances(sz, se2d, z_flat, embedding):
    return pl.pallas_call(
        _dist_body,
        grid=(_G,),
        in_specs=[
            pl.BlockSpec((_BM, 1), lambda i: (i, 0)),
            pl.BlockSpec((1, _K), lambda i: (0, 0)),
            pl.BlockSpec((_BM, _D), lambda i: (i, 0)),
            pl.BlockSpec((_K, _D), lambda i: (0, 0)),
        ],
        out_specs=[
            pl.BlockSpec((1, 1, _BM), lambda i: (i, 0, 0)),
            pl.BlockSpec((1, 1, _BM), lambda i: (i, 0, 0)),
        ],
        out_shape=[
            jax.ShapeDtypeStruct((_G, 1, _BM), jnp.int32),
            jax.ShapeDtypeStruct((_G, 1, _BM), jnp.float32),
        ],
    )(sz, se2d, z_flat, embedding)


@functools.lru_cache(maxsize=1)
def _make_sc_gather():
    info = plsc.get_sparse_core_info()
    nw = info.num_cores * info.num_subcores
    b_per_w = _N // nw
    ch = 128
    n_ch = b_per_w // ch
    mesh = plsc.VectorSubcoreMesh(core_axis_name="c", subcore_axis_name="s")

    @functools.partial(
        pl.kernel, mesh=mesh,
        out_type=jax.ShapeDtypeStruct((_N, _D), jnp.float32),
        scratch_types=[
            pltpu.VMEM((ch,), jnp.int32),
            pltpu.VMEM((ch, _D), jnp.float32),
            pltpu.SemaphoreType.DMA,
        ],
    )
    def gather_rows(emb_hbm, codes_hbm, out_hbm, idx_v, rows_v, sem):
        wid = lax.axis_index("s") * info.num_cores + lax.axis_index("c")
        base = wid * b_per_w
        for c in range(n_ch):
            off = base + c * ch
            pltpu.sync_copy(codes_hbm.at[pl.ds(off, ch)], idx_v)
            pltpu.async_copy(emb_hbm.at[idx_v], rows_v, sem).wait()
            pltpu.sync_copy(rows_v, out_hbm.at[pl.ds(off, ch)])

    return gather_rows


def kernel(z, embedding):
    z_flat = z.reshape(-1, _D)
    sz = jnp.sum(z_flat ** 2, axis=1, keepdims=True)
    se = jnp.sum(embedding ** 2, axis=1)
    codes3, minv3 = _argmin_distances(
        sz, se.reshape(1, _K), z_flat, embedding.astype(jnp.bfloat16))
    codes = codes3.reshape(_N)
    z_q = _make_sc_gather()(embedding, codes).reshape(z.shape)
    vq_loss = _COMMITMENT_COST * (jnp.sum(minv3) / (_N * _D))
    return (vq_loss, z_q, codes.reshape(z.shape[0], -1))


# speedup vs baseline: 1.3344x; 1.0554x over previous
import functools

import jax
import jax.numpy as jnp
from jax import lax
from jax.experimental import pallas as pl
from jax.experimental.pallas import tpu as pltpu
from jax.experimental.pallas import tpu_sc as plsc

_K = 8192
_D = 256
_N = 16384
_BM = 1024
_G = _N // _BM
_COMMITMENT_COST = 0.25


_W = 2736


def _dist_body(sz_ref, se_ref, z_ref, emb_ref, codes_ref, minv_ref):
    z = z_ref[...]
    e = emb_ref[...]
    mm = lax.dot_general(
        z.astype(jnp.bfloat16), e,
        dimension_numbers=(((1,), (1,)), ((), ())),
        preferred_element_type=jnp.float32)
    dist = (sz_ref[...] + se_ref[...]) - 2.0 * mm
    iota = lax.broadcasted_iota(jnp.int32, (_BM, _K), 1)
    acc_v = None
    acc_x = None
    acc_i = None
    for lo in range(0, _K, _W):
        hi = min(lo + _W, _K)
        dw = lax.slice(dist, (0, lo), (_BM, hi))
        iw = lax.slice(iota, (0, lo), (_BM, hi))
        m = jnp.min(dw, axis=1, keepdims=True)
        idx = jnp.min(jnp.where(dw == m, iw, _K), axis=1, keepdims=True)
        if acc_v is None:
            acc_v, acc_x, acc_i = m, m, idx
        else:
            keep = acc_v <= m
            acc_i = jnp.where(keep, acc_i, idx)
            acc_x = jnp.where(keep, acc_x, m)
            acc_v = jnp.where(keep, acc_v, m)
        acc_v = acc_v.astype(jnp.bfloat16).astype(jnp.float32)
    codes_ref[0, 0, :] = acc_i[:, 0]
    minv_ref[0, 0, :] = acc_x[:, 0]


def _argmin_distances(sz, se2d, z_flat, embedding):
    return pl.pallas_call(
        _dist_body,
        grid=(_G,),
        in_specs=[
            pl.BlockSpec((_BM, 1), lambda i: (i, 0)),
            pl.BlockSpec((1, _K), lambda i: (0, 0)),
            pl.BlockSpec((_BM, _D), lambda i: (i, 0)),
            pl.BlockSpec((_K, _D), lambda i: (0, 0)),
        ],
        out_specs=[
            pl.BlockSpec((1, 1, _BM), lambda i: (i, 0, 0)),
            pl.BlockSpec((1, 1, _BM), lambda i: (i, 0, 0)),
        ],
        out_shape=[
            jax.ShapeDtypeStruct((_G, 1, _BM), jnp.int32),
            jax.ShapeDtypeStruct((_G, 1, _BM), jnp.float32),
        ],
    )(sz, se2d, z_flat, embedding)


@functools.lru_cache(maxsize=1)
def _make_sc_gather():
    info = plsc.get_sparse_core_info()
    nw = info.num_cores * info.num_subcores
    b_per_w = _N // nw
    ch = 128
    n_ch = b_per_w // ch
    mesh = plsc.VectorSubcoreMesh(core_axis_name="c", subcore_axis_name="s")

    @functools.partial(
        pl.kernel, mesh=mesh,
        out_type=jax.ShapeDtypeStruct((_N, _D), jnp.float32),
        scratch_types=[
            pltpu.VMEM((ch,), jnp.int32),
            pltpu.VMEM((ch, _D), jnp.float32),
            pltpu.SemaphoreType.DMA,
        ],
    )
    def gather_rows(emb_hbm, codes_hbm, out_hbm, idx_v, rows_v, sem):
        wid = lax.axis_index("s") * info.num_cores + lax.axis_index("c")
        base = wid * b_per_w
        for c in range(n_ch):
            off = base + c * ch
            pltpu.sync_copy(codes_hbm.at[pl.ds(off, ch)], idx_v)
            pltpu.async_copy(emb_hbm.at[idx_v], rows_v, sem).wait()
            pltpu.sync_copy(rows_v, out_hbm.at[pl.ds(off, ch)])

    return gather_rows


def kernel(z, embedding):
    z_flat = z.reshape(-1, _D)
    sz = jnp.sum(z_flat ** 2, axis=1, keepdims=True)
    se = jnp.sum(embedding ** 2, axis=1)
    codes3, minv3 = _argmin_distances(
        sz, se.reshape(1, _K), z_flat, embedding.astype(jnp.bfloat16))
    codes = codes3.reshape(_N)
    z_q = _make_sc_gather()(embedding, codes).reshape(z.shape)
    vq_loss = _COMMITMENT_COST * (jnp.sum(minv3) / (_N * _D))
    return (vq_loss, z_q, codes.reshape(z.shape[0], -1))
